# baseline (speedup 1.0000x reference)
import jax
import jax.numpy as jnp
from jax.experimental import pallas as pl
from jax.experimental.pallas import tpu as pltpu
from jax.experimental.pallas import tpu_sc as plsc

IN_C, OUT_C = 32, 6
FEAT = 576
G, NMAX, K = 512, 40, 8
DIMS = [FEAT, 512, 256, 128, 64]

GB = 16
ROWS = GB * NMAX
NPROG = G // GB

SROWS = 64
HW = 1024 * 1024
PSPLIT = 4
PROWS = 256
CGRP = 8


def _stats_kernel(nf_ref, out_ref):
    i = pl.program_id(0)
    x = nf_ref[...].reshape(SROWS * NMAX, FEAT)
    s = jnp.sum(x, axis=0, keepdims=True)
    sq = jnp.sum(x * x, axis=0, keepdims=True)

    @pl.when(i == 0)
    def _():
        out_ref[0:1, :] = s
        out_ref[1:2, :] = sq

    @pl.when(i > 0)
    def _():
        out_ref[0:1, :] = out_ref[0:1, :] + s
        out_ref[1:2, :] = out_ref[1:2, :] + sq

    @pl.when(i == pl.num_programs(0) - 1)
    def _():
        n = float(G * NMAX)
        mean = out_ref[0:1, :] / n
        var = out_ref[1:2, :] / n - mean * mean
        out_ref[0:1, :] = mean
        out_ref[1:2, :] = jax.lax.rsqrt(var + 1e-5)


def _gcn_kernel(knn_ref, nf_ref, a_ref, stats_ref,
                w1_ref, b1_ref, w2_ref, b2_ref,
                w3_ref, b3_ref, w4_ref, b4_ref,
                x4_ref, idx_ref):
    mean = stats_ref[0:1, :]
    rstd = stats_ref[1:2, :]
    x = (nf_ref[...].reshape(ROWS, FEAT) - mean) * rstd
    A = a_ref[...]

    def layer(x, w_ref, b_ref, d_out):
        z = jax.lax.dot(x, w_ref[...])
        zb = z[:, d_out:]
        aggs = [jax.lax.dot(A[g], zb[g * NMAX:(g + 1) * NMAX, :])
                for g in range(GB)]
        agg = jnp.concatenate(aggs, axis=0)
        return jnp.maximum(z[:, :d_out] + agg + b_ref[...], 0.0)

    x = layer(x, w1_ref, b1_ref, DIMS[1])
    x = layer(x, w2_ref, b2_ref, DIMS[2])
    x = layer(x, w3_ref, b3_ref, DIMS[3])
    x = layer(x, w4_ref, b4_ref, DIMS[4])
    x4_ref[...] = jnp.concatenate(
        [x, jnp.zeros((ROWS, 128 - DIMS[4]), jnp.float32)], axis=1)

    gids = jax.lax.broadcasted_iota(jnp.int32, (GB, K), 0)
    idx_ref[0] = knn_ref[0] + (pl.program_id(0) * GB + gids) * NMAX


_NC, _NS = 2, 16
_NW = _NC * _NS
_BPW = (G * K) // _NW


def _sc_gather_kernel(table_ref, idx_ref, out_ref, idx_v, rows_v, sem):
    wid = jax.lax.axis_index("s") * _NC + jax.lax.axis_index("c")
    base = wid * _BPW
    pltpu.sync_copy(idx_ref.at[pl.ds(base, _BPW)], idx_v)
    pltpu.async_copy(table_ref.at[idx_v], rows_v, sem).wait()
    pltpu.sync_copy(rows_v, out_ref.at[pl.ds(base, _BPW)])


def _sc_gather(table, idx):
    k = pl.kernel(
        _sc_gather_kernel,
        mesh=plsc.VectorSubcoreMesh(core_axis_name="c", subcore_axis_name="s"),
        out_type=jax.ShapeDtypeStruct((G * K, 128), jnp.float32),
        scratch_types=[
            pltpu.VMEM((_BPW,), jnp.int32),
            pltpu.VMEM((_BPW, 128), jnp.float32),
            pltpu.SemaphoreType.DMA,
        ],
    )
    return k(table, idx)


def _cls_kernel(ef_ref, cw1_ref, cb1_ref, pa_ref, cw2_ref, cb2_ref, out_ref):
    h = jax.lax.dot(ef_ref[:, :DIMS[4]], cw1_ref[...]) + cb1_ref[...]
    h = jnp.where(h >= 0, h, pa_ref[...] * h)
    out_ref[...] = jax.lax.dot(h, cw2_ref[...]) + cb2_ref[...]


def _conv_kernel(w_ref, b_ref, x_ref, out_ref):
    c = pl.program_id(1)
    x = x_ref[0, 0]

    @pl.when(c == 0)
    def _():
        for o in range(OUT_C):
            out_ref[0, o] = b_ref[o, 0] + w_ref[o, 0] * x

    @pl.when(c > 0)
    def _():
        for o in range(OUT_C):
            out_ref[0, o] = out_ref[0, o] + w_ref[o, c] * x


def kernel(inputs, node_feats, adjacent_matrices, knn_inds, gt_labels,
           conv_w, conv_b,
           gcn_w1, gcn_b1, gcn_w2, gcn_b2, gcn_w3, gcn_b3, gcn_w4, gcn_b4,
           cls_w1, cls_b1, prelu_a, cls_w2, cls_b2):
    f32 = jnp.float32

    stats = pl.pallas_call(
        _stats_kernel,
        grid=(G // SROWS,),
        in_specs=[pl.BlockSpec((SROWS, NMAX, FEAT), lambda i: (i, 0, 0))],
        out_specs=pl.BlockSpec((2, FEAT), lambda i: (0, 0)),
        out_shape=jax.ShapeDtypeStruct((2, FEAT), f32),
    )(node_feats)

    w_cats = [jnp.concatenate([w[:d], w[d:]], axis=1)
              for w, d in ((gcn_w1, DIMS[0]), (gcn_w2, DIMS[1]),
                           (gcn_w3, DIMS[2]), (gcn_w4, DIMS[3]))]
    biases = [gcn_b1.reshape(1, -1), gcn_b2.reshape(1, -1),
              gcn_b3.reshape(1, -1), gcn_b4.reshape(1, -1)]

    full = lambda shape: pl.BlockSpec(shape, lambda i: tuple(0 for _ in shape))
    gcn_in_specs = [
        pl.BlockSpec((1, GB, K), lambda i: (i, 0, 0)),
        pl.BlockSpec((GB, NMAX, FEAT), lambda i: (i, 0, 0)),
        pl.BlockSpec((GB, NMAX, NMAX), lambda i: (i, 0, 0)),
        full((2, FEAT)),
    ]
    for li in range(4):
        d_in, d_out = DIMS[li], DIMS[li + 1]
        gcn_in_specs += [full((d_in, 2 * d_out)), full((1, d_out))]

    x4, idxs = pl.pallas_call(
        _gcn_kernel,
        grid=(NPROG,),
        in_specs=gcn_in_specs,
        out_specs=[pl.BlockSpec((ROWS, 128), lambda i: (i, 0)),
                   pl.BlockSpec((1, GB, K), lambda i: (i, 0, 0))],
        out_shape=[jax.ShapeDtypeStruct((G * NMAX, 128), f32),
                   jax.ShapeDtypeStruct((NPROG, GB, K), jnp.int32)],
    )(knn_inds.reshape(NPROG, GB, K), node_feats, adjacent_matrices, stats,
      w_cats[0], biases[0], w_cats[1], biases[1],
      w_cats[2], biases[2], w_cats[3], biases[3])

    ef = _sc_gather(x4, idxs.reshape(G * K))

    gcn_pred = pl.pallas_call(
        _cls_kernel,
        out_shape=jax.ShapeDtypeStruct((G * K, 2), f32),
    )(ef, cls_w1, cls_b1.reshape(1, -1), prelu_a.reshape(1, -1),
      cls_w2, cls_b2.reshape(1, -1))

    pred_maps = pl.pallas_call(
        _conv_kernel,
        grid=(4, IN_C),
        in_specs=[
            pl.BlockSpec(memory_space=pltpu.SMEM),
            pl.BlockSpec(memory_space=pltpu.SMEM),
            pl.BlockSpec((1, 1, 1024, 1024), lambda b, c: (b, c, 0, 0)),
        ],
        out_specs=pl.BlockSpec((1, OUT_C, 1024, 1024),
                               lambda b, c: (b, 0, 0, 0)),
        out_shape=jax.ShapeDtypeStruct((4, OUT_C, 1024, 1024), f32),
        compiler_params=pltpu.CompilerParams(vmem_limit_bytes=100 * 1024 * 1024),
    )(conv_w, conv_b.reshape(OUT_C, 1), inputs)

    return (pred_maps, gcn_pred, gt_labels)


# speedup vs baseline: 1.0584x; 1.0584x over previous
import jax
import jax.numpy as jnp
from jax.experimental import pallas as pl
from jax.experimental.pallas import tpu as pltpu
from jax.experimental.pallas import tpu_sc as plsc

IN_C, OUT_C = 32, 6
FEAT = 576
G, NMAX, K = 512, 40, 8
DIMS = [FEAT, 512, 256, 128, 64]

GB = 16
ROWS = GB * NMAX
NPROG = G // GB

SROWS = 64
HW = 1024 * 1024
PSPLIT = 4
PROWS = 256
CGRP = 8


def _stats_kernel(nf_ref, out_ref):
    i = pl.program_id(0)
    x = nf_ref[...].reshape(SROWS * NMAX, FEAT)
    s = jnp.sum(x, axis=0, keepdims=True)
    sq = jnp.sum(x * x, axis=0, keepdims=True)

    @pl.when(i == 0)
    def _():
        out_ref[0:1, :] = s
        out_ref[1:2, :] = sq

    @pl.when(i > 0)
    def _():
        out_ref[0:1, :] = out_ref[0:1, :] + s
        out_ref[1:2, :] = out_ref[1:2, :] + sq

    @pl.when(i == pl.num_programs(0) - 1)
    def _():
        n = float(G * NMAX)
        mean = out_ref[0:1, :] / n
        var = out_ref[1:2, :] / n - mean * mean
        out_ref[0:1, :] = mean
        out_ref[1:2, :] = jax.lax.rsqrt(var + 1e-5)


def _gcn_kernel(knn_ref, nf_ref, a_ref, stats_ref,
                w1_ref, b1_ref, w2_ref, b2_ref,
                w3_ref, b3_ref, w4_ref, b4_ref,
                x4_ref, idx_ref):
    mean = stats_ref[0:1, :]
    rstd = stats_ref[1:2, :]
    x = (nf_ref[...].reshape(ROWS, FEAT) - mean) * rstd
    A = a_ref[...]

    def layer(x, w_ref, b_ref, d_out):
        z = jax.lax.dot(x, w_ref[...])
        zb = z[:, d_out:]
        aggs = [jax.lax.dot(A[g], zb[g * NMAX:(g + 1) * NMAX, :])
                for g in range(GB)]
        agg = jnp.concatenate(aggs, axis=0)
        return jnp.maximum(z[:, :d_out] + agg + b_ref[...], 0.0)

    x = layer(x, w1_ref, b1_ref, DIMS[1])
    x = layer(x, w2_ref, b2_ref, DIMS[2])
    x = layer(x, w3_ref, b3_ref, DIMS[3])
    x = layer(x, w4_ref, b4_ref, DIMS[4])
    x4_ref[...] = jnp.concatenate(
        [x, jnp.zeros((ROWS, 128 - DIMS[4]), jnp.float32)], axis=1)

    gids = jax.lax.broadcasted_iota(jnp.int32, (GB, K), 0)
    idx_ref[0] = knn_ref[0] + (pl.program_id(0) * GB + gids) * NMAX


_NC, _NS = 2, 16
_NW = _NC * _NS
_BPW = (G * K) // _NW


def _sc_gather_kernel(table_ref, idx_ref, out_ref, idx_v, rows_v, sem):
    wid = jax.lax.axis_index("s") * _NC + jax.lax.axis_index("c")
    base = wid * _BPW
    pltpu.sync_copy(idx_ref.at[pl.ds(base, _BPW)], idx_v)
    pltpu.async_copy(table_ref.at[idx_v], rows_v, sem).wait()
    pltpu.sync_copy(rows_v, out_ref.at[pl.ds(base, _BPW)])


def _sc_gather(table, idx):
    k = pl.kernel(
        _sc_gather_kernel,
        mesh=plsc.VectorSubcoreMesh(core_axis_name="c", subcore_axis_name="s"),
        out_type=jax.ShapeDtypeStruct((G * K, 128), jnp.float32),
        scratch_types=[
            pltpu.VMEM((_BPW,), jnp.int32),
            pltpu.VMEM((_BPW, 128), jnp.float32),
            pltpu.SemaphoreType.DMA,
        ],
    )
    return k(table, idx)


def _cls_kernel(ef_ref, cw1_ref, cb1_ref, pa_ref, cw2_ref, cb2_ref, out_ref):
    h = jax.lax.dot(ef_ref[:, :DIMS[4]], cw1_ref[...]) + cb1_ref[...]
    h = jnp.where(h >= 0, h, pa_ref[...] * h)
    out_ref[...] = jax.lax.dot(h, cw2_ref[...]) + cb2_ref[...]


def _conv_kernel(w_ref, b_ref, *refs):
    cg = pl.program_id(2)
    out_ref = refs[-1]
    xs = [r[0, 0, 0] for r in refs[:CGRP]]

    @pl.when(cg == 0)
    def _():
        for o in range(OUT_C):
            acc = b_ref[o, 0] + w_ref[o, 0] * xs[0]
            for j in range(1, CGRP):
                acc = acc + w_ref[o, j] * xs[j]
            out_ref[0, 0, o] = acc

    @pl.when(cg > 0)
    def _():
        for o in range(OUT_C):
            acc = out_ref[0, 0, o]
            for j in range(CGRP):
                acc = acc + w_ref[o, cg * CGRP + j] * xs[j]
            out_ref[0, 0, o] = acc


def kernel(inputs, node_feats, adjacent_matrices, knn_inds, gt_labels,
           conv_w, conv_b,
           gcn_w1, gcn_b1, gcn_w2, gcn_b2, gcn_w3, gcn_b3, gcn_w4, gcn_b4,
           cls_w1, cls_b1, prelu_a, cls_w2, cls_b2):
    f32 = jnp.float32

    stats = pl.pallas_call(
        _stats_kernel,
        grid=(G // SROWS,),
        in_specs=[pl.BlockSpec((SROWS, NMAX, FEAT), lambda i: (i, 0, 0))],
        out_specs=pl.BlockSpec((2, FEAT), lambda i: (0, 0)),
        out_shape=jax.ShapeDtypeStruct((2, FEAT), f32),
    )(node_feats)

    w_cats = [jnp.concatenate([w[:d], w[d:]], axis=1)
              for w, d in ((gcn_w1, DIMS[0]), (gcn_w2, DIMS[1]),
                           (gcn_w3, DIMS[2]), (gcn_w4, DIMS[3]))]
    biases = [gcn_b1.reshape(1, -1), gcn_b2.reshape(1, -1),
              gcn_b3.reshape(1, -1), gcn_b4.reshape(1, -1)]

    full = lambda shape: pl.BlockSpec(shape, lambda i: tuple(0 for _ in shape))
    gcn_in_specs = [
        pl.BlockSpec((1, GB, K), lambda i: (i, 0, 0)),
        pl.BlockSpec((GB, NMAX, FEAT), lambda i: (i, 0, 0)),
        pl.BlockSpec((GB, NMAX, NMAX), lambda i: (i, 0, 0)),
        full((2, FEAT)),
    ]
    for li in range(4):
        d_in, d_out = DIMS[li], DIMS[li + 1]
        gcn_in_specs += [full((d_in, 2 * d_out)), full((1, d_out))]

    x4, idxs = pl.pallas_call(
        _gcn_kernel,
        grid=(NPROG,),
        in_specs=gcn_in_specs,
        out_specs=[pl.BlockSpec((ROWS, 128), lambda i: (i, 0)),
                   pl.BlockSpec((1, GB, K), lambda i: (i, 0, 0))],
        out_shape=[jax.ShapeDtypeStruct((G * NMAX, 128), f32),
                   jax.ShapeDtypeStruct((NPROG, GB, K), jnp.int32)],
    )(knn_inds.reshape(NPROG, GB, K), node_feats, adjacent_matrices, stats,
      w_cats[0], biases[0], w_cats[1], biases[1],
      w_cats[2], biases[2], w_cats[3], biases[3])

    ef = _sc_gather(x4, idxs.reshape(G * K))

    gcn_pred = pl.pallas_call(
        _cls_kernel,
        out_shape=jax.ShapeDtypeStruct((G * K, 2), f32),
    )(ef, cls_w1, cls_b1.reshape(1, -1), prelu_a.reshape(1, -1),
      cls_w2, cls_b2.reshape(1, -1))

    xin = inputs.reshape(4, IN_C, PSPLIT, PROWS, 1024)
    xspecs = [
        pl.BlockSpec((1, 1, 1, PROWS, 1024),
                     lambda b, p, c, j=j: (b, c * CGRP + j, p, 0, 0))
        for j in range(CGRP)
    ]
    pred = pl.pallas_call(
        _conv_kernel,
        grid=(4, PSPLIT, IN_C // CGRP),
        in_specs=[
            pl.BlockSpec(memory_space=pltpu.SMEM),
            pl.BlockSpec(memory_space=pltpu.SMEM),
        ] + xspecs,
        out_specs=pl.BlockSpec((1, 1, OUT_C, PROWS, 1024),
                               lambda b, p, c: (b, p, 0, 0, 0)),
        out_shape=jax.ShapeDtypeStruct((4, PSPLIT, OUT_C, PROWS, 1024), f32),
    )(conv_w, conv_b.reshape(OUT_C, 1), *([xin] * CGRP))
    pred_maps = pred.transpose(0, 2, 1, 3, 4).reshape(4, OUT_C, 1024, 1024)

    return (pred_maps, gcn_pred, gt_labels)


# speedup vs baseline: 1.1991x; 1.1330x over previous
import jax
import jax.numpy as jnp
from jax.experimental import pallas as pl
from jax.experimental.pallas import tpu as pltpu
from jax.experimental.pallas import tpu_sc as plsc

IN_C, OUT_C = 32, 6
FEAT = 576
G, NMAX, K = 512, 40, 8
DIMS = [FEAT, 512, 256, 128, 64]

GB = 16
ROWS = GB * NMAX
NPROG = G // GB

SROWS = 64
HW = 1024 * 1024
PSPLIT = 4
PROWS = 256
CGRP = 8


def _stats_kernel(nf_ref, out_ref):
    i = pl.program_id(0)
    x = nf_ref[...].reshape(SROWS * NMAX, FEAT)
    s = jnp.sum(x, axis=0, keepdims=True)
    sq = jnp.sum(x * x, axis=0, keepdims=True)

    @pl.when(i == 0)
    def _():
        out_ref[0:1, :] = s
        out_ref[1:2, :] = sq

    @pl.when(i > 0)
    def _():
        out_ref[0:1, :] = out_ref[0:1, :] + s
        out_ref[1:2, :] = out_ref[1:2, :] + sq

    @pl.when(i == pl.num_programs(0) - 1)
    def _():
        n = float(G * NMAX)
        mean = out_ref[0:1, :] / n
        var = out_ref[1:2, :] / n - mean * mean
        out_ref[0:1, :] = mean
        out_ref[1:2, :] = jax.lax.rsqrt(var + 1e-5)


def _gcn_kernel(knn_ref, nf_ref, a_ref, stats_ref,
                w1_ref, b1_ref, w2_ref, b2_ref,
                w3_ref, b3_ref, w4_ref, b4_ref,
                x4_ref, idx_ref):
    mean = stats_ref[0:1, :]
    rstd = stats_ref[1:2, :]
    x = (nf_ref[...].reshape(ROWS, FEAT) - mean) * rstd
    A = a_ref[...]

    def layer(x, w_ref, b_ref, d_out):
        z = jax.lax.dot(x, w_ref[...])
        zb = z[:, d_out:]
        aggs = [jax.lax.dot(A[g], zb[g * NMAX:(g + 1) * NMAX, :])
                for g in range(GB)]
        agg = jnp.concatenate(aggs, axis=0)
        return jnp.maximum(z[:, :d_out] + agg + b_ref[...], 0.0)

    x = layer(x, w1_ref, b1_ref, DIMS[1])
    x = layer(x, w2_ref, b2_ref, DIMS[2])
    x = layer(x, w3_ref, b3_ref, DIMS[3])
    x = layer(x, w4_ref, b4_ref, DIMS[4])
    x4_ref[...] = jnp.concatenate(
        [x, jnp.zeros((ROWS, 128 - DIMS[4]), jnp.float32)], axis=1)

    gids = jax.lax.broadcasted_iota(jnp.int32, (GB, K), 0)
    idx_ref[0] = knn_ref[0] + (pl.program_id(0) * GB + gids) * NMAX


_NC, _NS = 2, 16
_NW = _NC * _NS
_BPW = (G * K) // _NW


def _sc_gather_kernel(table_ref, idx_ref, out_ref, idx_v, rows_v, sem):
    wid = jax.lax.axis_index("s") * _NC + jax.lax.axis_index("c")
    base = wid * _BPW
    pltpu.sync_copy(idx_ref.at[pl.ds(base, _BPW)], idx_v)
    pltpu.async_copy(table_ref.at[idx_v], rows_v, sem).wait()
    pltpu.sync_copy(rows_v, out_ref.at[pl.ds(base, _BPW)])


def _sc_gather(table, idx):
    k = pl.kernel(
        _sc_gather_kernel,
        mesh=plsc.VectorSubcoreMesh(core_axis_name="c", subcore_axis_name="s"),
        out_type=jax.ShapeDtypeStruct((G * K, 128), jnp.float32),
        scratch_types=[
            pltpu.VMEM((_BPW,), jnp.int32),
            pltpu.VMEM((_BPW, 128), jnp.float32),
            pltpu.SemaphoreType.DMA,
        ],
    )
    return k(table, idx)


def _cls_kernel(ef_ref, cw1_ref, cb1_ref, pa_ref, cw2_ref, cb2_ref, out_ref):
    h = jax.lax.dot(ef_ref[:, :DIMS[4]], cw1_ref[...]) + cb1_ref[...]
    h = jnp.where(h >= 0, h, pa_ref[...] * h)
    out_ref[...] = jax.lax.dot(h, cw2_ref[...]) + cb2_ref[...]


def _conv_kernel(w_ref, b_ref, *refs):
    cg = pl.program_id(2)
    out_ref = refs[-1]
    xs = [r[0, 0, 0] for r in refs[:CGRP]]

    @pl.when(cg == 0)
    def _():
        for o in range(OUT_C):
            acc = b_ref[o, 0] + w_ref[o, 0] * xs[0]
            for j in range(1, CGRP):
                acc = acc + w_ref[o, j] * xs[j]
            out_ref[0, o, 0] = acc

    @pl.when(cg > 0)
    def _():
        for o in range(OUT_C):
            acc = out_ref[0, o, 0]
            for j in range(CGRP):
                acc = acc + w_ref[o, cg * CGRP + j] * xs[j]
            out_ref[0, o, 0] = acc


def kernel(inputs, node_feats, adjacent_matrices, knn_inds, gt_labels,
           conv_w, conv_b,
           gcn_w1, gcn_b1, gcn_w2, gcn_b2, gcn_w3, gcn_b3, gcn_w4, gcn_b4,
           cls_w1, cls_b1, prelu_a, cls_w2, cls_b2):
    f32 = jnp.float32

    stats = pl.pallas_call(
        _stats_kernel,
        grid=(G // SROWS,),
        in_specs=[pl.BlockSpec((SROWS, NMAX, FEAT), lambda i: (i, 0, 0))],
        out_specs=pl.BlockSpec((2, FEAT), lambda i: (0, 0)),
        out_shape=jax.ShapeDtypeStruct((2, FEAT), f32),
    )(node_feats)

    w_cats = [jnp.concatenate([w[:d], w[d:]], axis=1)
              for w, d in ((gcn_w1, DIMS[0]), (gcn_w2, DIMS[1]),
                           (gcn_w3, DIMS[2]), (gcn_w4, DIMS[3]))]
    biases = [gcn_b1.reshape(1, -1), gcn_b2.reshape(1, -1),
              gcn_b3.reshape(1, -1), gcn_b4.reshape(1, -1)]

    full = lambda shape: pl.BlockSpec(shape, lambda i: tuple(0 for _ in shape))
    gcn_in_specs = [
        pl.BlockSpec((1, GB, K), lambda i: (i, 0, 0)),
        pl.BlockSpec((GB, NMAX, FEAT), lambda i: (i, 0, 0)),
        pl.BlockSpec((GB, NMAX, NMAX), lambda i: (i, 0, 0)),
        full((2, FEAT)),
    ]
    for li in range(4):
        d_in, d_out = DIMS[li], DIMS[li + 1]
        gcn_in_specs += [full((d_in, 2 * d_out)), full((1, d_out))]

    x4, idxs = pl.pallas_call(
        _gcn_kernel,
        grid=(NPROG,),
        in_specs=gcn_in_specs,
        out_specs=[pl.BlockSpec((ROWS, 128), lambda i: (i, 0)),
                   pl.BlockSpec((1, GB, K), lambda i: (i, 0, 0))],
        out_shape=[jax.ShapeDtypeStruct((G * NMAX, 128), f32),
                   jax.ShapeDtypeStruct((NPROG, GB, K), jnp.int32)],
    )(knn_inds.reshape(NPROG, GB, K), node_feats, adjacent_matrices, stats,
      w_cats[0], biases[0], w_cats[1], biases[1],
      w_cats[2], biases[2], w_cats[3], biases[3])

    ef = _sc_gather(x4, idxs.reshape(G * K))

    gcn_pred = pl.pallas_call(
        _cls_kernel,
        out_shape=jax.ShapeDtypeStruct((G * K, 2), f32),
    )(ef, cls_w1, cls_b1.reshape(1, -1), prelu_a.reshape(1, -1),
      cls_w2, cls_b2.reshape(1, -1))

    xin = inputs.reshape(4, IN_C, PSPLIT, PROWS, 1024)
    xspecs = [
        pl.BlockSpec((1, 1, 1, PROWS, 1024),
                     lambda b, p, c, j=j: (b, c * CGRP + j, p, 0, 0))
        for j in range(CGRP)
    ]
    pred = pl.pallas_call(
        _conv_kernel,
        grid=(4, PSPLIT, IN_C // CGRP),
        in_specs=[
            pl.BlockSpec(memory_space=pltpu.SMEM),
            pl.BlockSpec(memory_space=pltpu.SMEM),
        ] + xspecs,
        out_specs=pl.BlockSpec((1, OUT_C, 1, PROWS, 1024),
                               lambda b, p, c: (b, 0, p, 0, 0)),
        out_shape=jax.ShapeDtypeStruct((4, OUT_C, PSPLIT, PROWS, 1024), f32),
    )(conv_w, conv_b.reshape(OUT_C, 1), *([xin] * CGRP))
    pred_maps = pred.reshape(4, OUT_C, 1024, 1024)

    return (pred_maps, gcn_pred, gt_labels)
